# baseline (speedup 1.0000x reference)
import jax
import jax.numpy as jnp
from jax.experimental import pallas as pl
from jax.experimental.pallas import tpu as pltpu

_B, _N, _D, _L, _K = 4, 2048, 32, 128, 16
_NSTEPS = 4
_H = (0.2 - 0.0) / _NSTEPS


def _var_pde_kernel(x_bn_ref, tokens_ref, theta_ref, alpha_ref, out_ref):
    x = x_bn_ref[0]
    mu = jnp.mean(x, axis=1, keepdims=True)
    xc = x - mu
    var = jnp.sum(xc * xc, axis=1, keepdims=True) / (_L - 1)
    xn = xc / (jnp.sqrt(var) + 1e-6)
    corr = jax.lax.dot_general(
        xn, xn, (((1,), (1,)), ((), ())),
        preferred_element_type=jnp.float32) * (1.0 / (_L - 1))

    col = jax.lax.broadcasted_iota(jnp.int32, (_N, _N), 1)
    w = corr
    for _ in range(_K + 1):
        m = jnp.max(w, axis=1, keepdims=True)
        pos = jnp.min(jnp.where(w >= m, col, _N), axis=1, keepdims=True)
        sel = col == pos
        w = jnp.where(sel, -w - 10.0, w)
    a = jnp.where(w < -2.0, -(w + 10.0), 0.0)

    a = 0.5 * (a + a.T)
    row = jax.lax.broadcasted_iota(jnp.int32, (_N, _N), 0)
    a = a + jnp.where(row == col, 1.0, 0.0)
    deg = jnp.maximum(jnp.sum(a, axis=1, keepdims=True), 1e-6)
    dinv = jax.lax.rsqrt(deg)
    a = a * dinv * dinv.reshape(1, _N)

    alpha = alpha_ref[0, 0]
    th = theta_ref[:]

    def rhs(v):
        p = jnp.dot(a, v, preferred_element_type=jnp.float32)
        r = jax.lax.dot_general(
            p, th, (((1,), (1,)), ((), ())),
            preferred_element_type=jnp.float32)
        return alpha * (v - p) + r

    y = tokens_ref[0]
    for _ in range(_NSTEPS):
        k1 = rhs(y)
        k2 = rhs(y + (0.5 * _H) * k1)
        k3 = rhs(y + (0.5 * _H) * k2)
        k4 = rhs(y + _H * k3)
        y = y + (_H / 6.0) * (k1 + 2.0 * k2 + 2.0 * k3 + k4)
    out_ref[0] = jnp.maximum(y, 0.0)


@jax.jit
def kernel(tokens, x_bn, theta_W, alpha_raw):
    alpha = jnp.minimum(jax.nn.softplus(alpha_raw), 2.0).reshape(1, 1)
    return pl.pallas_call(
        _var_pde_kernel,
        grid=(_B,),
        in_specs=[
            pl.BlockSpec((1, _N, _L), lambda b: (b, 0, 0)),
            pl.BlockSpec((1, _N, _D), lambda b: (b, 0, 0)),
            pl.BlockSpec((_D, _D), lambda b: (0, 0)),
            pl.BlockSpec((1, 1), lambda b: (0, 0)),
        ],
        out_specs=pl.BlockSpec((1, _N, _D), lambda b: (b, 0, 0)),
        out_shape=jax.ShapeDtypeStruct((_B, _N, _D), jnp.float32),
        compiler_params=pltpu.CompilerParams(
            dimension_semantics=("parallel",)),
    )(x_bn, tokens, theta_W, alpha)


# speedup vs baseline: 1.0010x; 1.0010x over previous
import jax
import jax.numpy as jnp
from jax.experimental import pallas as pl
from jax.experimental.pallas import tpu as pltpu

_B, _N, _D, _L, _K = 4, 2048, 32, 128, 16
_NSTEPS = 4
_H = (0.2 - 0.0) / _NSTEPS


def _var_pde_kernel(x_bn_ref, tokens_ref, theta_ref, alpha_ref, out_ref):
    x = x_bn_ref[0]
    mu = jnp.mean(x, axis=1, keepdims=True)
    xc = x - mu
    var = jnp.sum(xc * xc, axis=1, keepdims=True) / (_L - 1)
    xn = xc / (jnp.sqrt(var) + 1e-6)
    corr = jax.lax.dot_general(
        xn, xn, (((1,), (1,)), ((), ())),
        preferred_element_type=jnp.float32) * (1.0 / (_L - 1))

    col = jax.lax.broadcasted_iota(jnp.int32, (_N, _N), 1)
    w = corr
    for _ in range(_K + 1):
        m = jnp.max(w, axis=1, keepdims=True)
        pos = jnp.min(jnp.where(w >= m, col, _N), axis=1, keepdims=True)
        sel = col == pos
        w = jnp.where(sel, -w - 10.0, w)
    a = jnp.where(w < -2.0, -(w + 10.0), 0.0)

    a = 0.5 * (a + a.T)
    row = jax.lax.broadcasted_iota(jnp.int32, (_N, _N), 0)
    a = a + jnp.where(row == col, 1.0, 0.0)
    deg = jnp.maximum(jnp.sum(a, axis=1, keepdims=True), 1e-6)
    dinv = jax.lax.rsqrt(deg)
    a = a * dinv * dinv.reshape(1, _N)

    alpha = alpha_ref[0, 0]
    th = theta_ref[:]
    ab = a.astype(jnp.bfloat16)

    def rhs(v):
        p = jnp.dot(ab, v.astype(jnp.bfloat16),
                    preferred_element_type=jnp.float32)
        r = jax.lax.dot_general(
            p, th, (((1,), (1,)), ((), ())),
            preferred_element_type=jnp.float32)
        return alpha * (v - p) + r

    y = tokens_ref[0]
    for _ in range(_NSTEPS):
        k1 = rhs(y)
        k2 = rhs(y + (0.5 * _H) * k1)
        k3 = rhs(y + (0.5 * _H) * k2)
        k4 = rhs(y + _H * k3)
        y = y + (_H / 6.0) * (k1 + 2.0 * k2 + 2.0 * k3 + k4)
    out_ref[0] = jnp.maximum(y, 0.0)


@jax.jit
def kernel(tokens, x_bn, theta_W, alpha_raw):
    alpha = jnp.minimum(jax.nn.softplus(alpha_raw), 2.0).reshape(1, 1)
    return pl.pallas_call(
        _var_pde_kernel,
        grid=(_B,),
        in_specs=[
            pl.BlockSpec((1, _N, _L), lambda b: (b, 0, 0)),
            pl.BlockSpec((1, _N, _D), lambda b: (b, 0, 0)),
            pl.BlockSpec((_D, _D), lambda b: (0, 0)),
            pl.BlockSpec((1, 1), lambda b: (0, 0)),
        ],
        out_specs=pl.BlockSpec((1, _N, _D), lambda b: (b, 0, 0)),
        out_shape=jax.ShapeDtypeStruct((_B, _N, _D), jnp.float32),
        compiler_params=pltpu.CompilerParams(
            dimension_semantics=("parallel",)),
    )(x_bn, tokens, theta_W, alpha)


# speedup vs baseline: 1.0339x; 1.0328x over previous
import jax
import jax.numpy as jnp
from jax.experimental import pallas as pl
from jax.experimental.pallas import tpu as pltpu

_B, _N, _D, _L, _K = 4, 2048, 32, 128, 16
_NSTEPS = 4
_H = (0.2 - 0.0) / _NSTEPS


def _var_pde_kernel(x_bn_ref, tokens_ref, theta_ref, alpha_ref, out_ref):
    x = x_bn_ref[0]
    mu = jnp.mean(x, axis=1, keepdims=True)
    xc = x - mu
    var = jnp.sum(xc * xc, axis=1, keepdims=True) / (_L - 1)
    xn = xc / (jnp.sqrt(var) + 1e-6)
    corr = jax.lax.dot_general(
        xn, xn, (((1,), (1,)), ((), ())),
        preferred_element_type=jnp.float32) * (1.0 / (_L - 1))

    col = jax.lax.broadcasted_iota(jnp.int32, (_N, _N), 1)
    w = corr
    for _ in range(_K + 1):
        pos = jnp.argmax(w, axis=1).reshape(_N, 1)
        sel = col == pos
        w = jnp.where(sel, -w - 10.0, w)
    a = jnp.where(w < -2.0, -(w + 10.0), 0.0)

    a = 0.5 * (a + a.T)
    row = jax.lax.broadcasted_iota(jnp.int32, (_N, _N), 0)
    a = a + jnp.where(row == col, 1.0, 0.0)
    deg = jnp.maximum(jnp.sum(a, axis=1, keepdims=True), 1e-6)
    dinv = jax.lax.rsqrt(deg)
    a = a * dinv * dinv.reshape(1, _N)

    alpha = alpha_ref[0, 0]
    th = theta_ref[:]
    def rhs(v):
        p = jnp.dot(a, v, preferred_element_type=jnp.float32)
        r = jax.lax.dot_general(
            p, th, (((1,), (1,)), ((), ())),
            preferred_element_type=jnp.float32)
        return alpha * (v - p) + r

    y = tokens_ref[0]
    for _ in range(_NSTEPS):
        k1 = rhs(y)
        k2 = rhs(y + (0.5 * _H) * k1)
        k3 = rhs(y + (0.5 * _H) * k2)
        k4 = rhs(y + _H * k3)
        y = y + (_H / 6.0) * (k1 + 2.0 * k2 + 2.0 * k3 + k4)
    out_ref[0] = jnp.maximum(y, 0.0)


@jax.jit
def kernel(tokens, x_bn, theta_W, alpha_raw):
    alpha = jnp.minimum(jax.nn.softplus(alpha_raw), 2.0).reshape(1, 1)
    return pl.pallas_call(
        _var_pde_kernel,
        grid=(_B,),
        in_specs=[
            pl.BlockSpec((1, _N, _L), lambda b: (b, 0, 0)),
            pl.BlockSpec((1, _N, _D), lambda b: (b, 0, 0)),
            pl.BlockSpec((_D, _D), lambda b: (0, 0)),
            pl.BlockSpec((1, 1), lambda b: (0, 0)),
        ],
        out_specs=pl.BlockSpec((1, _N, _D), lambda b: (b, 0, 0)),
        out_shape=jax.ShapeDtypeStruct((_B, _N, _D), jnp.float32),
        compiler_params=pltpu.CompilerParams(
            dimension_semantics=("parallel",)),
    )(x_bn, tokens, theta_W, alpha)


# speedup vs baseline: 1.0769x; 1.0416x over previous
import jax
import jax.numpy as jnp
from jax.experimental import pallas as pl
from jax.experimental.pallas import tpu as pltpu

_B, _N, _D, _L, _K = 4, 2048, 32, 128, 16
_NSTEPS = 4
_H = (0.2 - 0.0) / _NSTEPS
_NEG = -3.0e38


def _var_pde_kernel(x_bn_ref, tokens_ref, theta_ref, alpha_ref, out_ref):
    x = x_bn_ref[0]
    mu = jnp.mean(x, axis=1, keepdims=True)
    xc = x - mu
    var = jnp.sum(xc * xc, axis=1, keepdims=True) / (_L - 1)
    xn = xc / (jnp.sqrt(var) + 1e-6)
    corr = jax.lax.dot_general(
        xn, xn, (((1,), (1,)), ((), ())),
        preferred_element_type=jnp.float32) * (1.0 / (_L - 1))

    col = jax.lax.broadcasted_iota(jnp.int32, (_N, _N), 1)
    m_prev = jnp.full((_N, 1), 3.0e38, jnp.float32)
    cum = jnp.zeros((_N, 1), jnp.int32)
    t17 = jnp.full((_N, 1), 3.0e38, jnp.float32)
    s = jnp.zeros((_N, 1), jnp.int32)
    c_cross = jnp.zeros((_N, 1), jnp.int32)
    for _ in range(_K + 1):
        m = jnp.max(jnp.where(corr < m_prev, corr, _NEG),
                    axis=1, keepdims=True)
        c = jnp.sum((corr == m).astype(jnp.int32), axis=1, keepdims=True)
        crossed = (cum < _K + 1) & (cum + c >= _K + 1)
        t17 = jnp.where(crossed, m, t17)
        s = jnp.where(crossed, _K + 1 - cum, s)
        c_cross = jnp.where(crossed, c, c_cross)
        cum = cum + c
        m_prev = m

    straddle = c_cross > s

    def _tie_search(_):
        lo = jnp.zeros((_N, 1), jnp.int32)
        hi = jnp.full((_N, 1), _N - 1, jnp.int32)
        for _ in range(11):
            mid = (lo + hi) >> 1
            cnt = jnp.sum(((corr == t17) & (col <= mid)).astype(jnp.int32),
                          axis=1, keepdims=True)
            ok = cnt >= s
            hi = jnp.where(ok, mid, hi)
            lo = jnp.where(ok, lo, mid + 1)
        return hi

    tiecut = jax.lax.cond(
        jnp.any(straddle), _tie_search,
        lambda _: jnp.full((_N, 1), _N - 1, jnp.int32), None)
    tiecut = jnp.where(straddle, tiecut, _N - 1)

    keep = (corr > t17) | ((corr == t17) & (col <= tiecut))
    a = jnp.where(keep, corr, 0.0)

    a = 0.5 * (a + a.T)
    row = jax.lax.broadcasted_iota(jnp.int32, (_N, _N), 0)
    a = a + jnp.where(row == col, 1.0, 0.0)
    deg = jnp.maximum(jnp.sum(a, axis=1, keepdims=True), 1e-6)
    dinv = jax.lax.rsqrt(deg)
    a = a * dinv * dinv.reshape(1, _N)

    alpha = alpha_ref[0, 0]
    th = theta_ref[:]
    def rhs(v):
        p = jnp.dot(a, v, preferred_element_type=jnp.float32)
        r = jax.lax.dot_general(
            p, th, (((1,), (1,)), ((), ())),
            preferred_element_type=jnp.float32)
        return alpha * (v - p) + r

    y = tokens_ref[0]
    for _ in range(_NSTEPS):
        k1 = rhs(y)
        k2 = rhs(y + (0.5 * _H) * k1)
        k3 = rhs(y + (0.5 * _H) * k2)
        k4 = rhs(y + _H * k3)
        y = y + (_H / 6.0) * (k1 + 2.0 * k2 + 2.0 * k3 + k4)
    out_ref[0] = jnp.maximum(y, 0.0)


@jax.jit
def kernel(tokens, x_bn, theta_W, alpha_raw):
    alpha = jnp.minimum(jax.nn.softplus(alpha_raw), 2.0).reshape(1, 1)
    return pl.pallas_call(
        _var_pde_kernel,
        grid=(_B,),
        in_specs=[
            pl.BlockSpec((1, _N, _L), lambda b: (b, 0, 0)),
            pl.BlockSpec((1, _N, _D), lambda b: (b, 0, 0)),
            pl.BlockSpec((_D, _D), lambda b: (0, 0)),
            pl.BlockSpec((1, 1), lambda b: (0, 0)),
        ],
        out_specs=pl.BlockSpec((1, _N, _D), lambda b: (b, 0, 0)),
        out_shape=jax.ShapeDtypeStruct((_B, _N, _D), jnp.float32),
        compiler_params=pltpu.CompilerParams(
            dimension_semantics=("parallel",)),
    )(x_bn, tokens, theta_W, alpha)


# speedup vs baseline: 1.1630x; 1.0800x over previous
import jax
import jax.numpy as jnp
from jax.experimental import pallas as pl
from jax.experimental.pallas import tpu as pltpu

_B, _N, _D, _L, _K = 4, 2048, 32, 128, 16
_NSTEPS = 4
_H = (0.2 - 0.0) / _NSTEPS
_NEG = -3.0e38


def _var_pde_kernel(x_bn_ref, tokens_ref, theta_ref, alpha_ref, out_ref):
    x = x_bn_ref[0]
    mu = jnp.mean(x, axis=1, keepdims=True)
    xc = x - mu
    var = jnp.sum(xc * xc, axis=1, keepdims=True) / (_L - 1)
    xn = xc / (jnp.sqrt(var) + 1e-6)
    corr = jax.lax.dot_general(
        xn, xn, (((1,), (1,)), ((), ())),
        preferred_element_type=jnp.float32) * (1.0 / (_L - 1))

    col = jax.lax.broadcasted_iota(jnp.int32, (_N, _N), 1)
    m_prev = jnp.full((_N, 1), 3.0e38, jnp.float32)
    ms = []
    for _ in range(_K + 1):
        m_prev = jnp.max(jnp.where(corr < m_prev, corr, _NEG),
                         axis=1, keepdims=True)
        ms.append(m_prev)
    mtab = jnp.concatenate(ms, axis=1)
    tcol = jax.lax.broadcasted_iota(jnp.int32, (_N, _K + 1), 1)

    def m_at(t):
        return jnp.sum(jnp.where(tcol == t, mtab, 0.0),
                       axis=1, keepdims=True)

    def cnt_ge(v):
        return jnp.sum((corr >= v).astype(jnp.int32), axis=1, keepdims=True)

    lo = jnp.zeros((_N, 1), jnp.int32)
    hi = jnp.full((_N, 1), _K, jnp.int32)
    for _ in range(5):
        mid = (lo + hi) >> 1
        ok = cnt_ge(m_at(mid)) >= _K + 1
        hi = jnp.where(ok, mid, hi)
        lo = jnp.where(ok, lo, mid + 1)
    t17 = m_at(lo)
    cum_prev = jnp.where(lo > 0, cnt_ge(m_at(jnp.maximum(lo - 1, 0))), 0)
    s = _K + 1 - cum_prev
    c_cross = cnt_ge(t17) - cum_prev

    straddle = c_cross > s

    def _tie_search(_):
        lo = jnp.zeros((_N, 1), jnp.int32)
        hi = jnp.full((_N, 1), _N - 1, jnp.int32)
        for _ in range(11):
            mid = (lo + hi) >> 1
            cnt = jnp.sum(((corr == t17) & (col <= mid)).astype(jnp.int32),
                          axis=1, keepdims=True)
            ok = cnt >= s
            hi = jnp.where(ok, mid, hi)
            lo = jnp.where(ok, lo, mid + 1)
        return hi

    tiecut = jax.lax.cond(
        jnp.any(straddle), _tie_search,
        lambda _: jnp.full((_N, 1), _N - 1, jnp.int32), None)
    tiecut = jnp.where(straddle, tiecut, _N - 1)

    keep = (corr > t17) | ((corr == t17) & (col <= tiecut))
    a = jnp.where(keep, corr, 0.0)

    a = 0.5 * (a + a.T)
    row = jax.lax.broadcasted_iota(jnp.int32, (_N, _N), 0)
    a = a + jnp.where(row == col, 1.0, 0.0)
    deg = jnp.maximum(jnp.sum(a, axis=1, keepdims=True), 1e-6)
    dinv = jax.lax.rsqrt(deg)
    a = a * dinv * dinv.reshape(1, _N)

    alpha = alpha_ref[0, 0]
    th = theta_ref[:]
    def rhs(v):
        p = jnp.dot(a, v, preferred_element_type=jnp.float32)
        r = jax.lax.dot_general(
            p, th, (((1,), (1,)), ((), ())),
            preferred_element_type=jnp.float32)
        return alpha * (v - p) + r

    y = tokens_ref[0]
    for _ in range(_NSTEPS):
        k1 = rhs(y)
        k2 = rhs(y + (0.5 * _H) * k1)
        k3 = rhs(y + (0.5 * _H) * k2)
        k4 = rhs(y + _H * k3)
        y = y + (_H / 6.0) * (k1 + 2.0 * k2 + 2.0 * k3 + k4)
    out_ref[0] = jnp.maximum(y, 0.0)


@jax.jit
def kernel(tokens, x_bn, theta_W, alpha_raw):
    alpha = jnp.minimum(jax.nn.softplus(alpha_raw), 2.0).reshape(1, 1)
    return pl.pallas_call(
        _var_pde_kernel,
        grid=(_B,),
        in_specs=[
            pl.BlockSpec((1, _N, _L), lambda b: (b, 0, 0)),
            pl.BlockSpec((1, _N, _D), lambda b: (b, 0, 0)),
            pl.BlockSpec((_D, _D), lambda b: (0, 0)),
            pl.BlockSpec((1, 1), lambda b: (0, 0)),
        ],
        out_specs=pl.BlockSpec((1, _N, _D), lambda b: (b, 0, 0)),
        out_shape=jax.ShapeDtypeStruct((_B, _N, _D), jnp.float32),
        compiler_params=pltpu.CompilerParams(
            dimension_semantics=("parallel",)),
    )(x_bn, tokens, theta_W, alpha)
